# Initial kernel scaffold; baseline (speedup 1.0000x reference)
import jax
import jax.numpy as jnp
from jax.experimental import pallas as pl


def kernel(x, edge_index, W0, W1, W2, W3, W4, W5, W6, g0, g1, g2, g3, g4, g5, g6, b0, b1, b2, b3, b4, b5, b6, Wc, bc):
    raise NotImplementedError("write your pallas kernel here")


# speedup vs baseline: 14.5825x; 14.5825x over previous
import functools

import jax
import jax.numpy as jnp
from jax import lax
from jax.experimental import pallas as pl
from jax.experimental.pallas import tpu as pltpu

_V = 1024
_K = 4
_EC = 2048


def _mbuild_body(e_ref, m_ref):
    i = pl.program_id(0)
    src = e_ref[0, :]
    dst = e_ref[1, :]
    cols = lax.broadcasted_iota(jnp.int32, (_EC, _V), 1)
    oh_s = (cols == src[:, None]).astype(jnp.float32)
    oh_d = (cols == dst[:, None]).astype(jnp.float32)
    upd = lax.dot_general(oh_s, oh_d, (((0,), (0,)), ((), ())),
                          preferred_element_type=jnp.float32, precision=lax.Precision.HIGHEST)

    @pl.when(i == 0)
    def _():
        m_ref[...] = jnp.zeros_like(m_ref)

    m_ref[...] += upd


def _build_mt(edge_index):
    n_chunks = edge_index.shape[1] // _EC
    return pl.pallas_call(
        _mbuild_body,
        grid=(n_chunks,),
        in_specs=[pl.BlockSpec((2, _EC), lambda i: (0, i))],
        out_specs=pl.BlockSpec((_V, _V), lambda i: (0, 0)),
        out_shape=jax.ShapeDtypeStruct((_V, _V), jnp.float32),
    )(edge_index)


def _norm_body(m_ref, a_ref):
    mt = m_ref[...]
    dout = jnp.sum(mt, axis=1, keepdims=True)
    din = jnp.sum(mt, axis=0, keepdims=True)
    a_ref[...] = -(lax.rsqrt(jnp.maximum(dout, 1.0)) * mt
                   * lax.rsqrt(jnp.maximum(din, 1.0)))


def _normalize(mt):
    return pl.pallas_call(
        _norm_body,
        out_shape=jax.ShapeDtypeStruct((_V, _V), jnp.float32),
    )(mt)


def _layer_body(ac_ref, h_ref, a_ref, w_ref, y_ref, st_ref, *, first):
    b = pl.program_id(0)
    x0 = h_ref[0]
    if not first:
        x0 = jnp.maximum(ac_ref[:, 0:1] * x0 + ac_ref[:, 1:2], 0.0)
    amat = a_ref[...]
    x1 = jnp.dot(x0, amat, preferred_element_type=jnp.float32, precision=lax.Precision.HIGHEST)
    x2 = 2.0 * jnp.dot(x1, amat, preferred_element_type=jnp.float32, precision=lax.Precision.HIGHEST) - x0
    x3 = 2.0 * jnp.dot(x2, amat, preferred_element_type=jnp.float32, precision=lax.Precision.HIGHEST) - x1
    xs = jnp.concatenate([x0, x1, x2, x3], axis=0)
    y = jnp.dot(w_ref[...], xs, preferred_element_type=jnp.float32)
    y_ref[0] = y
    s = jnp.sum(y, axis=1, keepdims=True)
    q = jnp.sum(y * y, axis=1, keepdims=True)

    @pl.when(b == 0)
    def _():
        st_ref[...] = jnp.zeros_like(st_ref)

    st_ref[...] += jnp.concatenate([s, q], axis=1)


def _layer(h, amat, wr, ac, first):
    bsz, fin, _ = h.shape
    fout = wr.shape[0]
    return pl.pallas_call(
        functools.partial(_layer_body, first=first),
        grid=(bsz,),
        in_specs=[
            pl.BlockSpec((fin, 2), lambda b: (0, 0)),
            pl.BlockSpec((1, fin, _V), lambda b: (b, 0, 0)),
            pl.BlockSpec((_V, _V), lambda b: (0, 0)),
            pl.BlockSpec((fout, _K * fin), lambda b: (0, 0)),
        ],
        out_specs=[
            pl.BlockSpec((1, fout, _V), lambda b: (b, 0, 0)),
            pl.BlockSpec((fout, 2), lambda b: (0, 0)),
        ],
        out_shape=[
            jax.ShapeDtypeStruct((bsz, fout, _V), jnp.float32),
            jax.ShapeDtypeStruct((fout, 2), jnp.float32),
        ],
    )(ac, h, amat, wr)


_CC = 8


def _cls_body(h_ref, ac_ref, w_ref, bc_ref, o_ref):
    c = pl.program_id(0)

    @pl.when(c == 0)
    def _():
        o_ref[...] = jnp.broadcast_to(bc_ref[...], o_ref.shape)

    acc = jnp.zeros_like(o_ref)
    for j in range(_CC):
        hc = h_ref[:, j, :]
        hn = jnp.maximum(ac_ref[j, 0] * hc + ac_ref[j, 1], 0.0)
        acc += jnp.dot(hn, w_ref[j], preferred_element_type=jnp.float32)
    o_ref[...] += acc


def _classifier(h, ac, wc7, bc):
    bsz, nch, _ = h.shape
    ncls = wc7.shape[2]
    return pl.pallas_call(
        _cls_body,
        grid=(nch // _CC,),
        in_specs=[
            pl.BlockSpec((bsz, _CC, _V), lambda c: (0, c, 0)),
            pl.BlockSpec((_CC, 2), lambda c: (c, 0)),
            pl.BlockSpec((_CC, _V, ncls), lambda c: (c, 0, 0)),
            pl.BlockSpec((1, ncls), lambda c: (0, 0)),
        ],
        out_specs=pl.BlockSpec((bsz, ncls), lambda c: (0, 0)),
        out_shape=jax.ShapeDtypeStruct((bsz, ncls), jnp.float32),
    )(h, ac, wc7, bc[None, :])


def _fold_bn(st, g, b, n):
    s, q = st[:, 0], st[:, 1]
    m = s / n
    v = q / n - m * m
    scale = g * lax.rsqrt(v + 1e-5)
    shift = b - m * scale
    return jnp.stack([scale, shift], axis=1)


def kernel(x, edge_index, W0, W1, W2, W3, W4, W5, W6, g0, g1, g2, g3, g4, g5, g6, b0, b1, b2, b3, b4, b5, b6, Wc, bc):
    ws = [W0, W1, W2, W3, W4, W5, W6]
    gs = [g0, g1, g2, g3, g4, g5, g6]
    bs = [b0, b1, b2, b3, b4, b5, b6]

    mt = _build_mt(edge_index)
    amat = _normalize(mt)

    bsz = x.shape[0]
    n = float(bsz * _V)
    h = x
    ac = None
    for li, w in enumerate(ws):
        fin = h.shape[1]
        fout = w.shape[0]
        wr = w.reshape(fout, fin, _K).transpose(0, 2, 1).reshape(fout, _K * fin)
        if li == 0:
            ac = jnp.zeros((fin, 2), jnp.float32)
        h, st = _layer(h, amat, wr, ac, first=(li == 0))
        ac = _fold_bn(st, gs[li], bs[li], n)

    ncls = Wc.shape[0]
    wc7 = Wc.reshape(ncls, h.shape[1], _V).transpose(1, 2, 0)
    return _classifier(h, ac, wc7, bc)


# speedup vs baseline: 17.4261x; 1.1950x over previous
import functools

import jax
import jax.numpy as jnp
from jax import lax
from jax.experimental import pallas as pl
from jax.experimental.pallas import tpu as pltpu

_V = 1024
_K = 4
_EC = 2048
_HOP_PREC = lax.Precision.HIGHEST


def _mbuild_body(e_ref, m_ref):
    i = pl.program_id(0)
    src = e_ref[0, :]
    dst = e_ref[1, :]
    cols = lax.broadcasted_iota(jnp.int32, (_EC, _V), 1)
    oh_s = (cols == src[:, None]).astype(jnp.float32)
    oh_d = (cols == dst[:, None]).astype(jnp.float32)
    upd = lax.dot_general(oh_s, oh_d, (((0,), (0,)), ((), ())),
                          preferred_element_type=jnp.float32)

    @pl.when(i == 0)
    def _():
        m_ref[...] = jnp.zeros_like(m_ref)

    m_ref[...] += upd


def _build_mt(edge_index):
    n_chunks = edge_index.shape[1] // _EC
    return pl.pallas_call(
        _mbuild_body,
        grid=(n_chunks,),
        in_specs=[pl.BlockSpec((2, _EC), lambda i: (0, i))],
        out_specs=pl.BlockSpec((_V, _V), lambda i: (0, 0)),
        out_shape=jax.ShapeDtypeStruct((_V, _V), jnp.float32),
    )(edge_index)


def _norm_body(m_ref, a_ref):
    mt = m_ref[...]
    dout = jnp.sum(mt, axis=1, keepdims=True)
    din = jnp.sum(mt, axis=0, keepdims=True)
    a_ref[...] = -(lax.rsqrt(jnp.maximum(dout, 1.0)) * mt
                   * lax.rsqrt(jnp.maximum(din, 1.0)))


def _normalize(mt):
    return pl.pallas_call(
        _norm_body,
        out_shape=jax.ShapeDtypeStruct((_V, _V), jnp.float32),
    )(mt)


def _layer_body(ac_ref, h_ref, a_ref, w_ref, y_ref, st_ref, *, first):
    b = pl.program_id(0)
    x0 = h_ref[0]
    if not first:
        x0 = jnp.maximum(ac_ref[:, 0:1] * x0 + ac_ref[:, 1:2], 0.0)
    amat = a_ref[...]
    x1 = jnp.dot(x0, amat, preferred_element_type=jnp.float32, precision=_HOP_PREC)
    x2 = 2.0 * jnp.dot(x1, amat, preferred_element_type=jnp.float32, precision=_HOP_PREC) - x0
    x3 = 2.0 * jnp.dot(x2, amat, preferred_element_type=jnp.float32, precision=_HOP_PREC) - x1
    xs = jnp.concatenate([x0, x1, x2, x3], axis=0)
    y = jnp.dot(w_ref[...], xs, preferred_element_type=jnp.float32)
    y_ref[0] = y
    s = jnp.sum(y, axis=1, keepdims=True)
    q = jnp.sum(y * y, axis=1, keepdims=True)

    @pl.when(b == 0)
    def _():
        st_ref[...] = jnp.zeros_like(st_ref)

    st_ref[...] += jnp.concatenate([s, q], axis=1)


def _layer(h, amat, wr, ac, first):
    bsz, fin, _ = h.shape
    fout = wr.shape[0]
    return pl.pallas_call(
        functools.partial(_layer_body, first=first),
        grid=(bsz,),
        in_specs=[
            pl.BlockSpec((fin, 2), lambda b: (0, 0)),
            pl.BlockSpec((1, fin, _V), lambda b: (b, 0, 0)),
            pl.BlockSpec((_V, _V), lambda b: (0, 0)),
            pl.BlockSpec((fout, _K * fin), lambda b: (0, 0)),
        ],
        out_specs=[
            pl.BlockSpec((1, fout, _V), lambda b: (b, 0, 0)),
            pl.BlockSpec((fout, 2), lambda b: (0, 0)),
        ],
        out_shape=[
            jax.ShapeDtypeStruct((bsz, fout, _V), jnp.float32),
            jax.ShapeDtypeStruct((fout, 2), jnp.float32),
        ],
    )(ac, h, amat, wr)


_CC = 8


def _cls_body(h_ref, ac_ref, w_ref, bc_ref, o_ref):
    c = pl.program_id(0)

    @pl.when(c == 0)
    def _():
        o_ref[...] = jnp.broadcast_to(bc_ref[...], o_ref.shape)

    acc = jnp.zeros_like(o_ref)
    for j in range(_CC):
        hc = h_ref[:, j, :]
        hn = jnp.maximum(ac_ref[j, 0] * hc + ac_ref[j, 1], 0.0)
        acc += jnp.dot(hn, w_ref[j], preferred_element_type=jnp.float32)
    o_ref[...] += acc


def _classifier(h, ac, wc7, bc):
    bsz, nch, _ = h.shape
    ncls = wc7.shape[2]
    return pl.pallas_call(
        _cls_body,
        grid=(nch // _CC,),
        in_specs=[
            pl.BlockSpec((bsz, _CC, _V), lambda c: (0, c, 0)),
            pl.BlockSpec((_CC, 2), lambda c: (c, 0)),
            pl.BlockSpec((_CC, _V, ncls), lambda c: (c, 0, 0)),
            pl.BlockSpec((1, ncls), lambda c: (0, 0)),
        ],
        out_specs=pl.BlockSpec((bsz, ncls), lambda c: (0, 0)),
        out_shape=jax.ShapeDtypeStruct((bsz, ncls), jnp.float32),
    )(h, ac, wc7, bc[None, :])


def _fold_bn(st, g, b, n):
    s, q = st[:, 0], st[:, 1]
    m = s / n
    v = q / n - m * m
    scale = g * lax.rsqrt(v + 1e-5)
    shift = b - m * scale
    return jnp.stack([scale, shift], axis=1)


def kernel(x, edge_index, W0, W1, W2, W3, W4, W5, W6, g0, g1, g2, g3, g4, g5, g6, b0, b1, b2, b3, b4, b5, b6, Wc, bc):
    ws = [W0, W1, W2, W3, W4, W5, W6]
    gs = [g0, g1, g2, g3, g4, g5, g6]
    bs = [b0, b1, b2, b3, b4, b5, b6]

    mt = _build_mt(edge_index)
    amat = _normalize(mt)

    bsz = x.shape[0]
    n = float(bsz * _V)
    h = x
    ac = None
    for li, w in enumerate(ws):
        fin = h.shape[1]
        fout = w.shape[0]
        wr = w.reshape(fout, fin, _K).transpose(0, 2, 1).reshape(fout, _K * fin)
        if li == 0:
            ac = jnp.zeros((fin, 2), jnp.float32)
        h, st = _layer(h, amat, wr, ac, first=(li == 0))
        ac = _fold_bn(st, gs[li], bs[li], n)

    ncls = Wc.shape[0]
    wc7 = Wc.reshape(ncls, h.shape[1], _V).transpose(1, 2, 0)
    return _classifier(h, ac, wc7, bc)


# speedup vs baseline: 30.7596x; 1.7651x over previous
import functools

import jax
import jax.numpy as jnp
from jax import lax
from jax.experimental import pallas as pl
from jax.experimental.pallas import tpu as pltpu

_V = 1024
_K = 4
_EC = 2048


def _mbuild_body(e_ref, m_ref):
    i = pl.program_id(0)
    src = e_ref[0, :]
    dst = e_ref[1, :]
    cols = lax.broadcasted_iota(jnp.int32, (_EC, _V), 1)
    oh_s = (cols == src[:, None]).astype(jnp.float32)
    oh_d = (cols == dst[:, None]).astype(jnp.float32)
    upd = lax.dot_general(oh_s, oh_d, (((0,), (0,)), ((), ())),
                          preferred_element_type=jnp.float32)

    @pl.when(i == 0)
    def _():
        m_ref[...] = jnp.zeros_like(m_ref)

    m_ref[...] += upd


def _build_mt(edge_index):
    n_chunks = edge_index.shape[1] // _EC
    return pl.pallas_call(
        _mbuild_body,
        grid=(n_chunks,),
        in_specs=[pl.BlockSpec((2, _EC), lambda i: (0, i))],
        out_specs=pl.BlockSpec((_V, _V), lambda i: (0, 0)),
        out_shape=jax.ShapeDtypeStruct((_V, _V), jnp.float32),
    )(edge_index)


def _norm_body(m_ref, hi_ref, lo_ref):
    mt = m_ref[...]
    dout = jnp.sum(mt, axis=1, keepdims=True)
    din = jnp.sum(mt, axis=0, keepdims=True)
    amat = -(lax.rsqrt(jnp.maximum(dout, 1.0)) * mt
             * lax.rsqrt(jnp.maximum(din, 1.0)))
    hi = amat.astype(jnp.bfloat16)
    hi_ref[...] = hi
    lo_ref[...] = (amat - hi.astype(jnp.float32)).astype(jnp.bfloat16)


def _normalize(mt):
    return pl.pallas_call(
        _norm_body,
        out_shape=[
            jax.ShapeDtypeStruct((_V, _V), jnp.bfloat16),
            jax.ShapeDtypeStruct((_V, _V), jnp.bfloat16),
        ],
    )(mt)


def _mm3(x, a_hi, a_lo):
    x_hi = x.astype(jnp.bfloat16)
    x_lo = (x - x_hi.astype(jnp.float32)).astype(jnp.bfloat16)
    r = jnp.dot(x_hi, a_lo, preferred_element_type=jnp.float32)
    r += jnp.dot(x_lo, a_hi, preferred_element_type=jnp.float32)
    r += jnp.dot(x_hi, a_hi, preferred_element_type=jnp.float32)
    return r


def _layer_body(ac_ref, h_ref, hi_ref, lo_ref, w_ref, y_ref, st_ref, *, first):
    b = pl.program_id(0)
    x0 = h_ref[0]
    if not first:
        x0 = jnp.maximum(ac_ref[:, 0:1] * x0 + ac_ref[:, 1:2], 0.0)
    a_hi = hi_ref[...]
    a_lo = lo_ref[...]
    x1 = _mm3(x0, a_hi, a_lo)
    x2 = 2.0 * _mm3(x1, a_hi, a_lo) - x0
    x3 = 2.0 * _mm3(x2, a_hi, a_lo) - x1
    xs = jnp.concatenate([x0, x1, x2, x3], axis=0)
    y = jnp.dot(w_ref[...], xs, preferred_element_type=jnp.float32)
    y_ref[0] = y
    s = jnp.sum(y, axis=1, keepdims=True)
    q = jnp.sum(y * y, axis=1, keepdims=True)

    @pl.when(b == 0)
    def _():
        st_ref[...] = jnp.zeros_like(st_ref)

    st_ref[...] += jnp.concatenate([s, q], axis=1)


def _layer(h, a_hi, a_lo, wr, ac, first):
    bsz, fin, _ = h.shape
    fout = wr.shape[0]
    return pl.pallas_call(
        functools.partial(_layer_body, first=first),
        grid=(bsz,),
        in_specs=[
            pl.BlockSpec((fin, 2), lambda b: (0, 0)),
            pl.BlockSpec((1, fin, _V), lambda b: (b, 0, 0)),
            pl.BlockSpec((_V, _V), lambda b: (0, 0)),
            pl.BlockSpec((_V, _V), lambda b: (0, 0)),
            pl.BlockSpec((fout, _K * fin), lambda b: (0, 0)),
        ],
        out_specs=[
            pl.BlockSpec((1, fout, _V), lambda b: (b, 0, 0)),
            pl.BlockSpec((fout, 2), lambda b: (0, 0)),
        ],
        out_shape=[
            jax.ShapeDtypeStruct((bsz, fout, _V), jnp.float32),
            jax.ShapeDtypeStruct((fout, 2), jnp.float32),
        ],
    )(ac, h, a_hi, a_lo, wr)


_CC = 8


def _cls_body(h_ref, ac_ref, w_ref, bc_ref, o_ref):
    c = pl.program_id(0)

    @pl.when(c == 0)
    def _():
        o_ref[...] = jnp.broadcast_to(bc_ref[...], o_ref.shape)

    acc = jnp.zeros_like(o_ref)
    for j in range(_CC):
        hc = h_ref[:, j, :]
        hn = jnp.maximum(ac_ref[j, 0] * hc + ac_ref[j, 1], 0.0)
        acc += jnp.dot(hn, w_ref[j], preferred_element_type=jnp.float32)
    o_ref[...] += acc


def _classifier(h, ac, wc7, bc):
    bsz, nch, _ = h.shape
    ncls = wc7.shape[2]
    return pl.pallas_call(
        _cls_body,
        grid=(nch // _CC,),
        in_specs=[
            pl.BlockSpec((bsz, _CC, _V), lambda c: (0, c, 0)),
            pl.BlockSpec((_CC, 2), lambda c: (c, 0)),
            pl.BlockSpec((_CC, _V, ncls), lambda c: (c, 0, 0)),
            pl.BlockSpec((1, ncls), lambda c: (0, 0)),
        ],
        out_specs=pl.BlockSpec((bsz, ncls), lambda c: (0, 0)),
        out_shape=jax.ShapeDtypeStruct((bsz, ncls), jnp.float32),
    )(h, ac, wc7, bc[None, :])


def _fold_bn(st, g, b, n):
    s, q = st[:, 0], st[:, 1]
    m = s / n
    v = q / n - m * m
    scale = g * lax.rsqrt(v + 1e-5)
    shift = b - m * scale
    return jnp.stack([scale, shift], axis=1)


def kernel(x, edge_index, W0, W1, W2, W3, W4, W5, W6, g0, g1, g2, g3, g4, g5, g6, b0, b1, b2, b3, b4, b5, b6, Wc, bc):
    ws = [W0, W1, W2, W3, W4, W5, W6]
    gs = [g0, g1, g2, g3, g4, g5, g6]
    bs = [b0, b1, b2, b3, b4, b5, b6]

    mt = _build_mt(edge_index)
    a_hi, a_lo = _normalize(mt)

    bsz = x.shape[0]
    n = float(bsz * _V)
    h = x
    ac = None
    for li, w in enumerate(ws):
        fin = h.shape[1]
        fout = w.shape[0]
        wr = w.reshape(fout, fin, _K).transpose(0, 2, 1).reshape(fout, _K * fin)
        if li == 0:
            ac = jnp.zeros((fin, 2), jnp.float32)
        h, st = _layer(h, a_hi, a_lo, wr, ac, first=(li == 0))
        ac = _fold_bn(st, gs[li], bs[li], n)

    ncls = Wc.shape[0]
    wc7 = Wc.reshape(ncls, h.shape[1], _V).transpose(1, 2, 0)
    return _classifier(h, ac, wc7, bc)
